# baseline (speedup 1.0000x reference)
import jax
import jax.numpy as jnp
from jax.experimental import pallas as pl
from jax.experimental.pallas import tpu as pltpu

NUM_E = 1024
DIM = 64
COMMIT = 0.25
TT = 512
RS = 64
CH = 128
NCH = NUM_E // CH


def _vq_body(x_ref, embbf_ref, embt2_ref,
             enc_ref, qst_ref, loss_ref, perp_ref,
             hist_ref, e2_ref, sse_ref):
    step = pl.program_id(0)
    nblk = pl.num_programs(0)
    ntok = nblk * TT

    @pl.when(step == 0)
    def _init():
        hist_ref[...] = jnp.zeros_like(hist_ref)
        sse_ref[...] = jnp.zeros_like(sse_ref)
        embt2 = embt2_ref[...]
        e2_ref[...] = jnp.sum(0.25 * (embt2 * embt2), axis=0,
                              keepdims=True)

    embbf = embbf_ref[...]
    embt2 = embt2_ref[...]
    e2 = e2_ref[...]
    onesbf = jnp.ones((1, RS), jnp.bfloat16)

    hist_step = jnp.zeros((1, NUM_E), jnp.float32)
    sse_step = jnp.zeros((RS, 1), jnp.float32)
    for st in range(TT // RS):
        r0 = st * RS
        xs = x_ref[r0:r0 + RS, :]
        m2s = jax.lax.dot_general(xs, embt2, (((1,), (0,)), ((), ())),
                                  preferred_element_type=jnp.float32)
        x2s = jnp.sum(xs * xs, axis=1, keepdims=True)
        runmin = jnp.full((RS, CH), jnp.inf, jnp.float32)
        runk = jnp.zeros((RS, CH), jnp.int32)
        for k in range(NCH):
            mk = m2s[:, k * CH:(k + 1) * CH]
            dk = (x2s + e2[:, k * CH:(k + 1) * CH]) - mk
            lt = dk < runmin
            runk = jnp.where(lt, k, runk)
            runmin = jnp.minimum(runmin, dk)
        dmin = jnp.min(runmin, axis=1, keepdims=True)
        lane = jax.lax.broadcasted_iota(jnp.int32, (RS, CH), 1)
        jlane = runk * CH + lane
        cand = jnp.where(runmin == dmin, jlane, 2 * NUM_E)
        idx = jnp.min(cand, axis=1, keepdims=True)
        ohs = []
        for k in range(NCH):
            ohk = (lane + k * CH == idx).astype(jnp.float32)
            ohs.append(ohk)
        onehot = jnp.concatenate(ohs, axis=1)
        enc_ref[r0:r0 + RS, :] = onehot
        ohbf = onehot.astype(jnp.bfloat16)
        hist_step += jax.lax.dot_general(onesbf, ohbf,
                                         (((1,), (0,)), ((), ())),
                                         preferred_element_type=jnp.float32)
        q = jax.lax.dot_general(ohbf, embbf, (((1,), (0,)), ((), ())),
                                preferred_element_type=jnp.float32)
        qst_ref[r0:r0 + RS, :] = xs + (q - xs)
        sse_step += dmin

    hist_ref[...] += hist_step
    sse_ref[...] += sse_step

    @pl.when(step == nblk - 1)
    def _fini():
        loss_ref[0, 0] = (1.0 + COMMIT) * jnp.sum(sse_ref[...]) / (ntok * DIM)
        avg = hist_ref[...] * (1.0 / ntok)
        ent = jnp.sum(avg * jnp.log(avg + 1e-10))
        perp_ref[0, 0] = jnp.exp(-ent)


def kernel(inputs, embedding):
    batch, channels, times = inputs.shape
    ntok = batch * times
    nblk = ntok // TT
    x = jnp.transpose(inputs, (0, 2, 1)).reshape(ntok, channels)
    embt2 = 2.0 * embedding.T
    embbf = embedding.astype(jnp.bfloat16)

    enc, qst, loss, perp = pl.pallas_call(
        _vq_body,
        grid=(nblk,),
        in_specs=[
            pl.BlockSpec((TT, DIM), lambda i: (i, 0)),
            pl.BlockSpec((NUM_E, DIM), lambda i: (0, 0)),
            pl.BlockSpec((DIM, NUM_E), lambda i: (0, 0)),
        ],
        out_specs=[
            pl.BlockSpec((TT, NUM_E), lambda i: (i, 0)),
            pl.BlockSpec((TT, DIM), lambda i: (i, 0)),
            pl.BlockSpec(memory_space=pltpu.SMEM),
            pl.BlockSpec(memory_space=pltpu.SMEM),
        ],
        out_shape=[
            jax.ShapeDtypeStruct((ntok, NUM_E), jnp.float32),
            jax.ShapeDtypeStruct((ntok, DIM), jnp.float32),
            jax.ShapeDtypeStruct((1, 1), jnp.float32),
            jax.ShapeDtypeStruct((1, 1), jnp.float32),
        ],
        scratch_shapes=[
            pltpu.VMEM((1, NUM_E), jnp.float32),
            pltpu.VMEM((1, NUM_E), jnp.float32),
            pltpu.VMEM((RS, 1), jnp.float32),
        ],
        compiler_params=pltpu.CompilerParams(
            dimension_semantics=("arbitrary",)),
    )(x, embbf, embt2)

    quantized_st = jnp.transpose(qst.reshape(batch, times, channels),
                                 (0, 2, 1))
    return loss[0, 0], quantized_st, perp[0, 0], enc


# speedup vs baseline: 1.9370x; 1.9370x over previous
import jax
import jax.numpy as jnp
from jax.experimental import pallas as pl
from jax.experimental.pallas import tpu as pltpu

NUM_E = 1024
DIM = 64
COMMIT = 0.25
TT = 512
RS = 64
CH = 128
NCH = NUM_E // CH


def _vq_body(x_ref, embbf_ref, embt2_ref,
             enc_ref, qst_ref, loss_ref, perp_ref,
             hist_ref, e2_ref, sse_ref):
    step = pl.program_id(0)
    nblk = pl.num_programs(0)
    ntok = nblk * TT

    @pl.when(step == 0)
    def _init():
        hist_ref[...] = jnp.zeros_like(hist_ref)
        sse_ref[...] = jnp.zeros_like(sse_ref)
        embt2 = embt2_ref[...]
        e2_ref[...] = jnp.sum(0.25 * (embt2 * embt2), axis=0,
                              keepdims=True)

    embbf = embbf_ref[...]
    embt2 = embt2_ref[...]
    e2 = e2_ref[...]
    onesbf = jnp.ones((1, RS), jnp.bfloat16)

    xb = x_ref[...]
    m2 = jax.lax.dot_general(xb, embt2, (((1,), (0,)), ((), ())),
                             preferred_element_type=jnp.float32)
    x2 = jnp.sum(xb * xb, axis=1, keepdims=True)

    hist_step = jnp.zeros((1, NUM_E), jnp.float32)
    sse_step = jnp.zeros((RS, 1), jnp.float32)
    for st in range(TT // RS):
        r0 = st * RS
        xs = xb[r0:r0 + RS, :]
        m2s = m2[r0:r0 + RS, :]
        x2s = x2[r0:r0 + RS, :]
        runmin = jnp.full((RS, CH), jnp.inf, jnp.float32)
        runk = jnp.zeros((RS, CH), jnp.int32)
        for k in range(NCH):
            mk = m2s[:, k * CH:(k + 1) * CH]
            dk = (x2s + e2[:, k * CH:(k + 1) * CH]) - mk
            lt = dk < runmin
            runk = jnp.where(lt, k, runk)
            runmin = jnp.minimum(runmin, dk)
        dmin = jnp.min(runmin, axis=1, keepdims=True)
        lane = jax.lax.broadcasted_iota(jnp.int32, (RS, CH), 1)
        jlane = runk * CH + lane
        cand = jnp.where(runmin == dmin, jlane, 2 * NUM_E)
        idx = jnp.min(cand, axis=1, keepdims=True)
        ohs = []
        for k in range(NCH):
            ohk = (lane + k * CH == idx).astype(jnp.float32)
            ohs.append(ohk)
        onehot = jnp.concatenate(ohs, axis=1)
        enc_ref[r0:r0 + RS, :] = onehot
        ohbf = onehot.astype(jnp.bfloat16)
        hist_step += jax.lax.dot_general(onesbf, ohbf,
                                         (((1,), (0,)), ((), ())),
                                         preferred_element_type=jnp.float32)
        q = jax.lax.dot_general(ohbf, embbf, (((1,), (0,)), ((), ())),
                                preferred_element_type=jnp.float32)
        qst_ref[r0:r0 + RS, :] = xs + (q - xs)
        sse_step += dmin

    hist_ref[...] += hist_step
    sse_ref[...] += sse_step

    @pl.when(step == nblk - 1)
    def _fini():
        loss_ref[0, 0] = (1.0 + COMMIT) * jnp.sum(sse_ref[...]) / (ntok * DIM)
        avg = hist_ref[...] * (1.0 / ntok)
        ent = jnp.sum(avg * jnp.log(avg + 1e-10))
        perp_ref[0, 0] = jnp.exp(-ent)


def kernel(inputs, embedding):
    batch, channels, times = inputs.shape
    ntok = batch * times
    nblk = ntok // TT
    x = jnp.transpose(inputs, (0, 2, 1)).reshape(ntok, channels)
    embt2 = 2.0 * embedding.T
    embbf = embedding.astype(jnp.bfloat16)

    enc, qst, loss, perp = pl.pallas_call(
        _vq_body,
        grid=(nblk,),
        in_specs=[
            pl.BlockSpec((TT, DIM), lambda i: (i, 0)),
            pl.BlockSpec((NUM_E, DIM), lambda i: (0, 0)),
            pl.BlockSpec((DIM, NUM_E), lambda i: (0, 0)),
        ],
        out_specs=[
            pl.BlockSpec((TT, NUM_E), lambda i: (i, 0)),
            pl.BlockSpec((TT, DIM), lambda i: (i, 0)),
            pl.BlockSpec(memory_space=pltpu.SMEM),
            pl.BlockSpec(memory_space=pltpu.SMEM),
        ],
        out_shape=[
            jax.ShapeDtypeStruct((ntok, NUM_E), jnp.float32),
            jax.ShapeDtypeStruct((ntok, DIM), jnp.float32),
            jax.ShapeDtypeStruct((1, 1), jnp.float32),
            jax.ShapeDtypeStruct((1, 1), jnp.float32),
        ],
        scratch_shapes=[
            pltpu.VMEM((1, NUM_E), jnp.float32),
            pltpu.VMEM((1, NUM_E), jnp.float32),
            pltpu.VMEM((RS, 1), jnp.float32),
        ],
        compiler_params=pltpu.CompilerParams(
            dimension_semantics=("arbitrary",)),
    )(x, embbf, embt2)

    quantized_st = jnp.transpose(qst.reshape(batch, times, channels),
                                 (0, 2, 1))
    return loss[0, 0], quantized_st, perp[0, 0], enc


# speedup vs baseline: 2.2229x; 1.1477x over previous
import jax
import jax.numpy as jnp
from jax.experimental import pallas as pl
from jax.experimental.pallas import tpu as pltpu

NUM_E = 1024
DIM = 64
COMMIT = 0.25
TT = 512
RS = 64
CH = 128
NCH = NUM_E // CH


def _vq_body(x_ref, embbf_ref, embt2_ref,
             enc_ref, qst_ref, loss_ref, perp_ref,
             hist_ref, e2_ref, sse_ref):
    step = pl.program_id(0)
    nblk = pl.num_programs(0)
    ntok = nblk * TT

    @pl.when(step == 0)
    def _init():
        hist_ref[...] = jnp.zeros_like(hist_ref)
        sse_ref[...] = jnp.zeros_like(sse_ref)
        embt2 = embt2_ref[...]
        e2_ref[...] = jnp.sum(0.25 * (embt2 * embt2), axis=0,
                              keepdims=True)

    embbf = embbf_ref[...]
    embt2 = embt2_ref[...]
    e2 = e2_ref[...]
    onesbf = jnp.ones((1, RS), jnp.bfloat16)

    xb = x_ref[...]
    m2 = jax.lax.dot_general(xb, embt2, (((1,), (0,)), ((), ())),
                             preferred_element_type=jnp.float32)
    x2 = jnp.sum(xb * xb, axis=1, keepdims=True)

    hist_step = jnp.zeros((1, NUM_E), jnp.float32)
    sse_step = jnp.zeros((RS, 1), jnp.float32)
    for st in range(TT // RS):
        r0 = st * RS
        xs = xb[r0:r0 + RS, :]
        m2s = m2[r0:r0 + RS, :]
        x2s = x2[r0:r0 + RS, :]
        runmin = jnp.full((RS, CH), jnp.inf, jnp.float32)
        runk = jnp.zeros((RS, CH), jnp.int32)
        for k in range(NCH):
            mk = m2s[:, k * CH:(k + 1) * CH]
            dk = (x2s + e2[:, k * CH:(k + 1) * CH]) - mk
            lt = dk < runmin
            runk = jnp.where(lt, k, runk)
            runmin = jnp.minimum(runmin, dk)
        dmin = jnp.min(runmin, axis=1, keepdims=True)
        lane = jax.lax.broadcasted_iota(jnp.int32, (RS, CH), 1)
        jlane = runk * CH + lane
        cand = jnp.where(runmin == dmin, jlane, 2 * NUM_E)
        idx = jnp.min(cand, axis=1, keepdims=True)
        ohs = []
        for k in range(NCH):
            ohk = (lane + k * CH == idx).astype(jnp.float32)
            ohs.append(ohk)
        onehot = jnp.concatenate(ohs, axis=1)
        enc_ref[r0:r0 + RS, :] = onehot
        hist_step += jnp.sum(onehot, axis=0, keepdims=True)
        q = jax.lax.dot_general(onehot.astype(jnp.bfloat16), embbf,
                                (((1,), (0,)), ((), ())),
                                preferred_element_type=jnp.float32)
        qst_ref[r0:r0 + RS, :] = xs + (q - xs)
        sse_step += dmin

    hist_ref[...] += hist_step
    sse_ref[...] += sse_step

    @pl.when(step == nblk - 1)
    def _fini():
        loss_ref[0, 0] = (1.0 + COMMIT) * jnp.sum(sse_ref[...]) / (ntok * DIM)
        avg = hist_ref[...] * (1.0 / ntok)
        ent = jnp.sum(avg * jnp.log(avg + 1e-10))
        perp_ref[0, 0] = jnp.exp(-ent)


def kernel(inputs, embedding):
    batch, channels, times = inputs.shape
    ntok = batch * times
    nblk = ntok // TT
    x = jnp.transpose(inputs, (0, 2, 1)).reshape(ntok, channels)
    embt2 = 2.0 * embedding.T
    embbf = embedding.astype(jnp.bfloat16)

    enc, qst, loss, perp = pl.pallas_call(
        _vq_body,
        grid=(nblk,),
        in_specs=[
            pl.BlockSpec((TT, DIM), lambda i: (i, 0)),
            pl.BlockSpec((NUM_E, DIM), lambda i: (0, 0)),
            pl.BlockSpec((DIM, NUM_E), lambda i: (0, 0)),
        ],
        out_specs=[
            pl.BlockSpec((TT, NUM_E), lambda i: (i, 0)),
            pl.BlockSpec((TT, DIM), lambda i: (i, 0)),
            pl.BlockSpec(memory_space=pltpu.SMEM),
            pl.BlockSpec(memory_space=pltpu.SMEM),
        ],
        out_shape=[
            jax.ShapeDtypeStruct((ntok, NUM_E), jnp.float32),
            jax.ShapeDtypeStruct((ntok, DIM), jnp.float32),
            jax.ShapeDtypeStruct((1, 1), jnp.float32),
            jax.ShapeDtypeStruct((1, 1), jnp.float32),
        ],
        scratch_shapes=[
            pltpu.VMEM((1, NUM_E), jnp.float32),
            pltpu.VMEM((1, NUM_E), jnp.float32),
            pltpu.VMEM((RS, 1), jnp.float32),
        ],
        compiler_params=pltpu.CompilerParams(
            dimension_semantics=("arbitrary",)),
    )(x, embbf, embt2)

    quantized_st = jnp.transpose(qst.reshape(batch, times, channels),
                                 (0, 2, 1))
    return loss[0, 0], quantized_st, perp[0, 0], enc


# speedup vs baseline: 2.3654x; 1.0641x over previous
import jax
import jax.numpy as jnp
from jax.experimental import pallas as pl
from jax.experimental.pallas import tpu as pltpu

NUM_E = 1024
DIM = 64
COMMIT = 0.25
TT = 512
RS = 64
CH = 128
NCH = NUM_E // CH


def _vq_body(x_ref, embbf_ref, embt2_ref,
             enc_ref, qst_ref, loss_ref, perp_ref,
             hist_ref, e2_ref, sse_ref):
    step = pl.program_id(0)
    nblk = pl.num_programs(0)
    ntok = nblk * TT

    @pl.when(step == 0)
    def _init():
        hist_ref[...] = jnp.zeros_like(hist_ref)
        sse_ref[...] = jnp.zeros_like(sse_ref)
        embt2 = embt2_ref[...]
        e2_ref[...] = jnp.sum(0.25 * (embt2 * embt2), axis=0,
                              keepdims=True)

    embbf = embbf_ref[...]
    embt2 = embt2_ref[...]
    e2 = e2_ref[...]

    xcols = x_ref[0]
    xb = jnp.concatenate(
        [xcols[:, st * RS:(st + 1) * RS].T for st in range(TT // RS)],
        axis=0)
    m2 = jax.lax.dot_general(xb, embt2, (((1,), (0,)), ((), ())),
                             preferred_element_type=jnp.float32)
    x2 = jnp.sum(xb * xb, axis=1, keepdims=True)

    hist_step = jnp.zeros((1, NUM_E), jnp.float32)
    sse_step = jnp.zeros((RS, 1), jnp.float32)
    qts = []
    for st in range(TT // RS):
        r0 = st * RS
        xs = xb[r0:r0 + RS, :]
        m2s = m2[r0:r0 + RS, :]
        x2s = x2[r0:r0 + RS, :]
        runmin = jnp.full((RS, CH), jnp.inf, jnp.float32)
        runk = jnp.zeros((RS, CH), jnp.int32)
        for k in range(NCH):
            mk = m2s[:, k * CH:(k + 1) * CH]
            dk = (x2s + e2[:, k * CH:(k + 1) * CH]) - mk
            lt = dk < runmin
            runk = jnp.where(lt, k, runk)
            runmin = jnp.minimum(runmin, dk)
        dmin = jnp.min(runmin, axis=1, keepdims=True)
        lane = jax.lax.broadcasted_iota(jnp.int32, (RS, CH), 1)
        jlane = runk * CH + lane
        cand = jnp.where(runmin == dmin, jlane, 2 * NUM_E)
        idx = jnp.min(cand, axis=1, keepdims=True)
        ohs = []
        for k in range(NCH):
            ohk = (lane + k * CH == idx).astype(jnp.float32)
            ohs.append(ohk)
        onehot = jnp.concatenate(ohs, axis=1)
        enc_ref[r0:r0 + RS, :] = onehot
        hist_step += jnp.sum(onehot, axis=0, keepdims=True)
        q = jax.lax.dot_general(onehot.astype(jnp.bfloat16), embbf,
                                (((1,), (0,)), ((), ())),
                                preferred_element_type=jnp.float32)
        qts.append((xs + (q - xs)).T)
        sse_step += dmin

    qst_ref[0] = jnp.concatenate(qts, axis=1)
    hist_ref[...] += hist_step
    sse_ref[...] += sse_step

    @pl.when(step == nblk - 1)
    def _fini():
        loss_ref[0, 0] = (1.0 + COMMIT) * jnp.sum(sse_ref[...]) / (ntok * DIM)
        avg = hist_ref[...] * (1.0 / ntok)
        ent = jnp.sum(avg * jnp.log(avg + 1e-10))
        perp_ref[0, 0] = jnp.exp(-ent)


def kernel(inputs, embedding):
    batch, channels, times = inputs.shape
    ntok = batch * times
    nblk = ntok // TT
    tpb = times // TT
    embt2 = 2.0 * embedding.T
    embbf = embedding.astype(jnp.bfloat16)

    enc, qst, loss, perp = pl.pallas_call(
        _vq_body,
        grid=(nblk,),
        in_specs=[
            pl.BlockSpec((1, DIM, TT), lambda i: (i // tpb, 0, i % tpb)),
            pl.BlockSpec((NUM_E, DIM), lambda i: (0, 0)),
            pl.BlockSpec((DIM, NUM_E), lambda i: (0, 0)),
        ],
        out_specs=[
            pl.BlockSpec((TT, NUM_E), lambda i: (i, 0)),
            pl.BlockSpec((1, DIM, TT), lambda i: (i // tpb, 0, i % tpb)),
            pl.BlockSpec(memory_space=pltpu.SMEM),
            pl.BlockSpec(memory_space=pltpu.SMEM),
        ],
        out_shape=[
            jax.ShapeDtypeStruct((ntok, NUM_E), jnp.float32),
            jax.ShapeDtypeStruct((batch, channels, times), jnp.float32),
            jax.ShapeDtypeStruct((1, 1), jnp.float32),
            jax.ShapeDtypeStruct((1, 1), jnp.float32),
        ],
        scratch_shapes=[
            pltpu.VMEM((1, NUM_E), jnp.float32),
            pltpu.VMEM((1, NUM_E), jnp.float32),
            pltpu.VMEM((RS, 1), jnp.float32),
        ],
        compiler_params=pltpu.CompilerParams(
            dimension_semantics=("arbitrary",)),
    )(inputs, embbf, embt2)

    return loss[0, 0], qst, perp[0, 0], enc
